# Initial kernel scaffold; baseline (speedup 1.0000x reference)
import functools

import jax
import jax.numpy as jnp
from jax import lax
from jax.experimental import pallas as pl
from jax.experimental.pallas import tpu as pltpu
from jax.experimental.pallas import tpu_sc as plsc

C = 16
RES = 512
NC = 2
NS = 16
NW = NC * NS
B = 256
L = 16
PLANE_DIMS = ((1, 2), (0, 2), (0, 1))


def _floor_parts(coord):
    p = coord * RES - 0.5
    t = p.astype(jnp.int32)
    tf = t.astype(jnp.float32)
    neg = tf > p
    fl_i = jnp.where(neg, t - 1, t)
    fl_f = jnp.where(neg, tf - 1.0, tf)
    w = p - fl_f
    i0 = jnp.clip(fl_i, 0, RES - 1)
    i1 = jnp.minimum(i0 + 1, RES - 1)
    return i0, i1, w


def _splat(vec, lane):
    idx = jnp.full((L,), lane, jnp.int32)
    return jnp.take(vec, idx, mode="promise_in_bounds")


def _sc_body(n_pad, xt_hbm, fm_hbm, out_hbm, xb, idxb, wb, rows, outb, sem):
    per_w = n_pad // NW
    n_chunks = per_w // B
    wid = lax.axis_index("s") * NC + lax.axis_index("c")

    def chunk_body(k, _):
        base = wid * per_w + k * B

        for d in range(3):
            pltpu.sync_copy(xt_hbm.at[d, pl.ds(base, B)], xb.at[d])

        def idx_body(g, _):
            s = pl.ds(g * L, L)
            for plane, (ud, vd) in enumerate(PLANE_DIMS):
                u = xb[ud, s]
                v = xb[vd, s]
                x0, x1, wx = _floor_parts(u)
                y0, y1, wy = _floor_parts(v)
                pbase = plane * RES * RES
                r0 = pbase + (y0 << 9)
                r1 = pbase + (y1 << 9)
                idxb[4 * plane + 0, s] = r0 + x0
                idxb[4 * plane + 1, s] = r0 + x1
                idxb[4 * plane + 2, s] = r1 + x0
                idxb[4 * plane + 3, s] = r1 + x1
                wb[2 * plane + 0, s] = wx
                wb[2 * plane + 1, s] = wy
            return ()

        lax.fori_loop(0, B // L, idx_body, ())

        copies = []
        for tap in range(12):
            for h in range(B // 128):
                cp = pltpu.async_copy(
                    fm_hbm.at[idxb.at[tap, pl.ds(h * 128, 128)]],
                    rows.at[pl.ds(tap * B + h * 128, 128)],
                    sem,
                )
                copies.append(cp)
        for cp in copies:
            cp.wait()

        def blend_body(g, _):
            s = pl.ds(g * L, L)
            wv = [wb[i, s] for i in range(6)]
            for p in range(L):
                pt = g * L + p
                for plane in range(3):
                    wx = _splat(wv[2 * plane + 0], p)
                    wy = _splat(wv[2 * plane + 1], p)
                    f00 = rows[(4 * plane + 0) * B + pt, :]
                    f01 = rows[(4 * plane + 1) * B + pt, :]
                    f10 = rows[(4 * plane + 2) * B + pt, :]
                    f11 = rows[(4 * plane + 3) * B + pt, :]
                    top = f00 + wx * (f01 - f00)
                    bot = f10 + wx * (f11 - f10)
                    outb[pt, pl.ds(plane * C, C)] = top + wy * (bot - top)
            return ()

        lax.fori_loop(0, B // L, blend_body, ())

        pltpu.sync_copy(outb, out_hbm.at[pl.ds(base, B)])
        return ()

    lax.fori_loop(0, n_chunks, chunk_body, ())


@jax.jit
def kernel(x, fm):
    n = x.shape[0]
    per_w = -(-n // (NW * B)) * B
    n_pad = per_w * NW
    xt = jnp.zeros((3, n_pad), jnp.float32).at[:, :n].set(x.T)
    fm_flat = fm.reshape(3 * RES * RES, C)

    mesh = plsc.VectorSubcoreMesh(
        core_axis_name="c", subcore_axis_name="s", num_cores=NC, num_subcores=NS
    )
    out = pl.kernel(
        functools.partial(_sc_body, n_pad),
        out_type=jax.ShapeDtypeStruct((n_pad, 3 * C), jnp.float32),
        mesh=mesh,
        scratch_types=[
            pltpu.VMEM((3, B), jnp.float32),
            pltpu.VMEM((12, B), jnp.int32),
            pltpu.VMEM((6, B), jnp.float32),
            pltpu.VMEM((12 * B, C), jnp.float32),
            pltpu.VMEM((B, 3 * C), jnp.float32),
            pltpu.SemaphoreType.DMA,
        ],
    )(xt, fm_flat)
    return out[:n]


# speedup vs baseline: 43.4658x; 43.4658x over previous
import functools

import jax
import jax.numpy as jnp
from jax import lax
from jax.experimental import pallas as pl
from jax.experimental.pallas import tpu as pltpu
from jax.experimental.pallas import tpu_sc as plsc

C = 16
RES = 512
NC = 2
NS = 16
NW = NC * NS
B = 256
L = 16
PLANE_DIMS = ((1, 2), (0, 2), (0, 1))


def _floor_parts(coord):
    p = coord * RES - 0.5
    t = p.astype(jnp.int32)
    tf = t.astype(jnp.float32)
    neg = tf > p
    fl_i = jnp.where(neg, t - 1, t)
    fl_f = jnp.where(neg, tf - 1.0, tf)
    w = p - fl_f
    i0 = jnp.clip(fl_i, 0, RES - 1)
    i1 = jnp.minimum(i0 + 1, RES - 1)
    return i0, i1, w


def _splat(vec, lane):
    idx = jnp.full((L,), lane, jnp.int32)
    return jnp.take_along_axis(vec, idx, axis=0)


def _sc_body(n_pad, xt_hbm, fm_hbm, out_hbm, xb, idxb, wb, rows, outb, sem):
    per_w = n_pad // NW
    n_chunks = per_w // B
    wid = lax.axis_index("s") * NC + lax.axis_index("c")

    def chunk_body(k, _):
        base = wid * per_w + k * B

        for d in range(3):
            pltpu.sync_copy(
                xt_hbm.at[pl.ds(d * n_pad + base, B)], xb.at[pl.ds(d * B, B)]
            )

        def idx_body(g, _):
            o = g * L
            for plane, (ud, vd) in enumerate(PLANE_DIMS):
                u = xb[pl.ds(ud * B + o, L)]
                v = xb[pl.ds(vd * B + o, L)]
                x0, x1, wx = _floor_parts(u)
                y0, y1, wy = _floor_parts(v)
                pbase = plane * RES * RES
                r0 = pbase + (y0 << 9)
                r1 = pbase + (y1 << 9)
                idxb[pl.ds((4 * plane + 0) * B + o, L)] = r0 + x0
                idxb[pl.ds((4 * plane + 1) * B + o, L)] = r0 + x1
                idxb[pl.ds((4 * plane + 2) * B + o, L)] = r1 + x0
                idxb[pl.ds((4 * plane + 3) * B + o, L)] = r1 + x1
                wb[pl.ds((2 * plane + 0) * B + o, L)] = wx
                wb[pl.ds((2 * plane + 1) * B + o, L)] = wy
            return ()

        lax.fori_loop(0, B // L, idx_body, ())

        copies = []
        for tap in range(12):
            for h in range(B // 128):
                cp = pltpu.async_copy(
                    fm_hbm.at[idxb.at[pl.ds(tap * B + h * 128, 128)]],
                    rows.at[pl.ds(tap * B + h * 128, 128)],
                    sem,
                )
                copies.append(cp)
        for cp in copies:
            cp.wait()

        def blend_body(g, _):
            o = g * L
            wv = [wb[pl.ds(i * B + o, L)] for i in range(6)]
            for p in range(L):
                pt = o + p
                for plane in range(3):
                    wx = _splat(wv[2 * plane + 0], p)
                    wy = _splat(wv[2 * plane + 1], p)
                    f00 = rows[(4 * plane + 0) * B + pt, :]
                    f01 = rows[(4 * plane + 1) * B + pt, :]
                    f10 = rows[(4 * plane + 2) * B + pt, :]
                    f11 = rows[(4 * plane + 3) * B + pt, :]
                    top = f00 + wx * (f01 - f00)
                    bot = f10 + wx * (f11 - f10)
                    outb[pl.ds(pt * 3 * C + plane * C, C)] = (
                        top + wy * (bot - top)
                    )
            return ()

        lax.fori_loop(0, B // L, blend_body, ())

        pltpu.sync_copy(outb, out_hbm.at[pl.ds(base * 3 * C, B * 3 * C)])
        return ()

    lax.fori_loop(0, n_chunks, chunk_body, ())


@jax.jit
def kernel(x, fm):
    n = x.shape[0]
    per_w = -(-n // (NW * B)) * B
    n_pad = per_w * NW
    xt = jnp.zeros((3, n_pad), jnp.float32).at[:, :n].set(x.T).reshape(-1)
    fm_flat = fm.reshape(3 * RES * RES, C)

    mesh = plsc.VectorSubcoreMesh(
        core_axis_name="c", subcore_axis_name="s", num_cores=NC, num_subcores=NS
    )
    out = pl.kernel(
        functools.partial(_sc_body, n_pad),
        out_type=jax.ShapeDtypeStruct((n_pad * 3 * C,), jnp.float32),
        mesh=mesh,
        scratch_types=[
            pltpu.VMEM((3 * B,), jnp.float32),
            pltpu.VMEM((12 * B,), jnp.int32),
            pltpu.VMEM((6 * B,), jnp.float32),
            pltpu.VMEM((12 * B, C), jnp.float32),
            pltpu.VMEM((B * 3 * C,), jnp.float32),
            pltpu.SemaphoreType.DMA,
        ],
        compiler_params=pltpu.CompilerParams(use_tc_tiling_on_sc=False),
    )(xt, fm_flat)
    return out.reshape(n_pad, 3 * C)[:n]
